# baseline (speedup 1.0000x reference)
import functools

import jax
import jax.numpy as jnp
from jax import lax
from jax.experimental import pallas as pl
from jax.experimental.pallas import tpu as pltpu
from jax.experimental.pallas import tpu_sc as plsc

_EPS = 1e-12

F, H, W, C = 64, 32, 32, 768
F_SC = 12
F_TC = F - F_SC
FB = 4

TROW = 6 * 8 * 128
TPF = (H * W) // 8
TRW = 4
WCHUNK = TRW * TROW


def _sc_body(g_hbm, out_hbm, bufs, acc, sems):
    wid = lax.axis_index("s") * 2 + lax.axis_index("c")
    tr0 = wid * TRW

    def start(step, slot):
        pltpu.async_copy(
            g_hbm.at[pl.ds(F_TC + step, 1), pl.ds(tr0, TRW)],
            bufs.at[slot], sems.at[slot])

    def wait(slot):
        pltpu.make_async_copy(g_hbm.at[pl.ds(0, 1), pl.ds(0, TRW)],
                              bufs.at[slot], sems.at[slot]).wait()

    def accum(step, slot):
        def add_init(t):
            @plsc.parallel_loop(0, TROW // 16, unroll=8)
            def _(i):
                acc[pl.ds(t * TROW + i * 16, 16)] = bufs[slot, 0, t,
                                                         pl.ds(i * 16, 16)]

        def add_acc(t):
            @plsc.parallel_loop(0, TROW // 16, unroll=8)
            def _(i):
                o = t * TROW + i * 16
                acc[pl.ds(o, 16)] = acc[pl.ds(o, 16)] + bufs[slot, 0, t,
                                                             pl.ds(i * 16, 16)]

        @pl.when(step == 0)
        def _():
            for t in range(TRW):
                add_init(t)

        @pl.when(step > 0)
        def _():
            for t in range(TRW):
                add_acc(t)

    start(jnp.int32(0), 0)

    def step_fn(step, _):
        even = lax.rem(step, 2) == 0

        @pl.when(even)
        def _():
            wait(0)

            @pl.when(step + 1 < F_SC)
            def _():
                start(step + 1, 1)
            accum(step, 0)

        @pl.when(jnp.logical_not(even))
        def _():
            wait(1)

            @pl.when(step + 1 < F_SC)
            def _():
                start(step + 1, 0)
            accum(step, 1)

        return 0

    lax.fori_loop(0, F_SC, step_fn, 0)

    pltpu.sync_copy(acc, out_hbm.at[pl.ds(tr0 * TROW, WCHUNK)])


def _sc_part(g_tiled):
    mesh = plsc.VectorSubcoreMesh(core_axis_name="c", subcore_axis_name="s",
                                  num_cores=2, num_subcores=16)
    k = functools.partial(
        pl.kernel,
        mesh=mesh,
        out_type=jax.ShapeDtypeStruct((TPF * TROW,), jnp.float32),
        scratch_types=[
            pltpu.VMEM((2, 1, TRW, TROW), jnp.float32),
            pltpu.VMEM((WCHUNK,), jnp.float32),
            pltpu.SemaphoreType.DMA((2,)),
        ],
    )(_sc_body)
    return k(g_tiled)


def _tc_sum_body(g_ref, out_ref, acc_ref):
    f = pl.program_id(0)
    s = ((g_ref[0] + g_ref[1]) + (g_ref[2] + g_ref[3]))

    @pl.when(f == 0)
    def _():
        acc_ref[...] = s

    @pl.when(f > 0)
    def _():
        acc_ref[...] += s

    @pl.when(f == F_TC // FB - 1)
    def _():
        out_ref[...] = acc_ref[...]


def _tc_sum(g):
    return pl.pallas_call(
        _tc_sum_body,
        grid=(F_TC // FB,),
        in_specs=[pl.BlockSpec((FB, H, W, C), lambda f: (f, 0, 0, 0))],
        out_specs=pl.BlockSpec((H, W, C), lambda f: (0, 0, 0)),
        out_shape=jax.ShapeDtypeStruct((H, W, C), jnp.float32),
        scratch_shapes=[pltpu.VMEM((H, W, C), jnp.float32)],
        compiler_params=pltpu.CompilerParams(
            dimension_semantics=("arbitrary",),
        ),
    )(g)


def _combine_body(a_ref, b_ref, row_ref, col_ref, tte_ref, w_ref, bias_ref,
                  out_ref):
    m = (a_ref[...] + b_ref[...]) * (1.0 / F)
    emb = (m + row_ref[...][:, None, :] + col_ref[...][None, :, :]
           + tte_ref[...][None, :, :])
    mu = jnp.mean(emb, axis=-1, keepdims=True)
    d = emb - mu
    var = jnp.mean(d * d, axis=-1, keepdims=True)
    y = d * jax.lax.rsqrt(var + _EPS)
    out_ref[...] = y * w_ref[...][None, None, :] + bias_ref[...][None, None, :]


def _combine(a, b, row_emb, col_emb, tte, lnw, lnb):
    return pl.pallas_call(
        _combine_body,
        out_shape=jax.ShapeDtypeStruct((H, W, C), jnp.float32),
    )(a, b, row_emb, col_emb, tte, lnw, lnb)


def kernel(grid, row_emb, col_emb, token_type_emb, ln_weight, ln_bias):
    B = grid.shape[0]
    g = grid.reshape(F, H, W, C)
    g_tiled = (grid.reshape(F * TPF, 8, 6, 128)
               .transpose(0, 2, 1, 3)
               .reshape(F, TPF, TROW))
    sc_sum_tiled = _sc_part(g_tiled)
    sc_sum = (sc_sum_tiled.reshape(TPF, 6, 8, 128)
              .transpose(0, 2, 1, 3)
              .reshape(H, W, C))
    tc_sum = _tc_sum(g)
    out = _combine(tc_sum, sc_sum, row_emb, col_emb, token_type_emb,
                   ln_weight, ln_bias)
    return out.reshape(B, H * W, C)


# speedup vs baseline: 3.2305x; 3.2305x over previous
import functools

import jax
import jax.numpy as jnp
from jax import lax
from jax.experimental import pallas as pl
from jax.experimental.pallas import tpu as pltpu
from jax.experimental.pallas import tpu_sc as plsc

_EPS = 1e-12

F, H, W, C = 64, 32, 32, 768
F_SC = 12
F_TC = F - F_SC
FB = 4
RPW = (H * W) // 32
CVROW = C // 16


def _sc_body(g_hbm, out_hbm, bufs, acc, sems):
    wid = lax.axis_index("s") * 2 + lax.axis_index("c")
    r0 = wid * RPW

    def start(step, slot):
        f = F_TC + step
        pltpu.async_copy(g_hbm.at[pl.ds(f * (H * W) + r0, RPW)],
                         bufs.at[slot], sems.at[slot])

    def wait(slot):
        pltpu.make_async_copy(g_hbm.at[pl.ds(0, RPW)],
                              bufs.at[slot], sems.at[slot]).wait()

    def accum(step, slot):
        def add_init(r):
            @plsc.parallel_loop(0, CVROW, unroll=8)
            def _(j):
                sl = pl.ds(j * 16, 16)
                acc[r, sl] = bufs[slot, r, sl]

        def add_acc(r):
            @plsc.parallel_loop(0, CVROW, unroll=8)
            def _(j):
                sl = pl.ds(j * 16, 16)
                acc[r, sl] = acc[r, sl] + bufs[slot, r, sl]

        @pl.when(step == 0)
        def _():
            for r in range(RPW):
                add_init(r)

        @pl.when(step > 0)
        def _():
            for r in range(RPW):
                add_acc(r)

    start(jnp.int32(0), 0)

    def step_fn(step, _):
        even = lax.rem(step, 2) == 0

        @pl.when(even)
        def _():
            wait(0)

            @pl.when(step + 1 < F_SC)
            def _():
                start(step + 1, 1)
            accum(step, 0)

        @pl.when(jnp.logical_not(even))
        def _():
            wait(1)

            @pl.when(step + 1 < F_SC)
            def _():
                start(step + 1, 0)
            accum(step, 1)

        return 0

    lax.fori_loop(0, F_SC, step_fn, 0)

    pltpu.sync_copy(acc, out_hbm.at[pl.ds(r0, RPW)])


def _sc_part(g2d):
    mesh = plsc.VectorSubcoreMesh(core_axis_name="c", subcore_axis_name="s",
                                  num_cores=2, num_subcores=16)
    k = functools.partial(
        pl.kernel,
        mesh=mesh,
        out_type=jax.ShapeDtypeStruct((H * W, C), jnp.float32),
        scratch_types=[
            pltpu.VMEM((2, RPW, C), jnp.float32),
            pltpu.VMEM((RPW, C), jnp.float32),
            pltpu.SemaphoreType.DMA((2,)),
        ],
        compiler_params=pltpu.CompilerParams(use_tc_tiling_on_sc=True),
    )(_sc_body)
    return k(g2d)


def _tc_sum_body(g_ref, out_ref, acc_ref):
    f = pl.program_id(0)
    s = ((g_ref[0] + g_ref[1]) + (g_ref[2] + g_ref[3]))

    @pl.when(f == 0)
    def _():
        acc_ref[...] = s

    @pl.when(f > 0)
    def _():
        acc_ref[...] += s

    @pl.when(f == F_TC // FB - 1)
    def _():
        out_ref[...] = acc_ref[...]


def _tc_sum(g):
    return pl.pallas_call(
        _tc_sum_body,
        grid=(F_TC // FB,),
        in_specs=[pl.BlockSpec((FB, H, W, C), lambda f: (f, 0, 0, 0))],
        out_specs=pl.BlockSpec((H, W, C), lambda f: (0, 0, 0)),
        out_shape=jax.ShapeDtypeStruct((H, W, C), jnp.float32),
        scratch_shapes=[pltpu.VMEM((H, W, C), jnp.float32)],
        compiler_params=pltpu.CompilerParams(
            dimension_semantics=("arbitrary",),
        ),
    )(g)


def _combine_body(a_ref, b_ref, row_ref, col_ref, tte_ref, w_ref, bias_ref,
                  out_ref):
    m = (a_ref[...] + b_ref[...]) * (1.0 / F)
    emb = (m + row_ref[...][:, None, :] + col_ref[...][None, :, :]
           + tte_ref[...][None, :, :])
    mu = jnp.mean(emb, axis=-1, keepdims=True)
    d = emb - mu
    var = jnp.mean(d * d, axis=-1, keepdims=True)
    y = d * jax.lax.rsqrt(var + _EPS)
    out_ref[...] = y * w_ref[...][None, None, :] + bias_ref[...][None, None, :]


def _combine(a, b, row_emb, col_emb, tte, lnw, lnb):
    return pl.pallas_call(
        _combine_body,
        out_shape=jax.ShapeDtypeStruct((H, W, C), jnp.float32),
    )(a, b, row_emb, col_emb, tte, lnw, lnb)


def kernel(grid, row_emb, col_emb, token_type_emb, ln_weight, ln_bias):
    B = grid.shape[0]
    g = grid.reshape(F, H, W, C)
    sc_sum = _sc_part(grid.reshape(F * H * W, C)).reshape(H, W, C)
    tc_sum = _tc_sum(g)
    out = _combine(tc_sum, sc_sum, row_emb, col_emb, token_type_emb,
                   ln_weight, ln_bias)
    return out.reshape(B, H * W, C)


# speedup vs baseline: 4.2734x; 1.3228x over previous
import functools

import jax
import jax.numpy as jnp
from jax.experimental import pallas as pl
from jax.experimental.pallas import tpu as pltpu

_EPS = 1e-12


_FB = 4


def _body(g_ref, row_ref, col_ref, tte_ref, w_ref, b_ref, out_ref, acc_ref,
          *, num_steps, num_frames):
    f = pl.program_id(0)
    s = ((g_ref[0] + g_ref[1]) + (g_ref[2] + g_ref[3]))

    @pl.when(f == 0)
    def _init():
        acc_ref[...] = s

    @pl.when(f > 0)
    def _accum():
        acc_ref[...] += s

    @pl.when(f == num_steps - 1)
    def _finish():
        m = acc_ref[...] * (1.0 / num_frames)
        emb = (m + row_ref[...][:, None, :] + col_ref[...][None, :, :]
               + tte_ref[...][None, :, :])
        mu = jnp.mean(emb, axis=-1, keepdims=True)
        d = emb - mu
        var = jnp.mean(d * d, axis=-1, keepdims=True)
        y = d * jax.lax.rsqrt(var + _EPS)
        out_ref[...] = y * w_ref[...][None, None, :] + b_ref[...][None, None, :]


def kernel(grid, row_emb, col_emb, token_type_emb, ln_weight, ln_bias):
    B, F, H, W, C = grid.shape
    g = grid.reshape(F, H, W, C)

    out = pl.pallas_call(
        functools.partial(_body, num_steps=F // _FB, num_frames=F),
        grid=(F // _FB,),
        in_specs=[
            pl.BlockSpec((_FB, H, W, C), lambda f: (f, 0, 0, 0)),
            pl.BlockSpec((H, C), lambda f: (0, 0)),
            pl.BlockSpec((W, C), lambda f: (0, 0)),
            pl.BlockSpec((1, C), lambda f: (0, 0)),
            pl.BlockSpec((C,), lambda f: (0,)),
            pl.BlockSpec((C,), lambda f: (0,)),
        ],
        out_specs=pl.BlockSpec((H, W, C), lambda f: (0, 0, 0)),
        out_shape=jax.ShapeDtypeStruct((H, W, C), jnp.float32),
        scratch_shapes=[pltpu.VMEM((H, W, C), jnp.float32)],
        compiler_params=pltpu.CompilerParams(
            dimension_semantics=("arbitrary",),
        ),
    )(g, row_emb, col_emb, token_type_emb, ln_weight, ln_bias)

    return out.reshape(B, H * W, C)
